# baseline (speedup 1.0000x reference)
import functools

import jax
import jax.numpy as jnp
from jax import lax
from jax.experimental import pallas as pl
from jax.experimental.pallas import tpu as pltpu
from jax.experimental.pallas import tpu_sc as plsc

_INTERPRET = False

_NEG = float("-inf")


def _main_body(h_ref, W1_ref, b1_ref, Wa_ref, ba_ref, Wb_ref, bb_ref,
               wc_ref, bc_ref, Wcls_ref, bcls_ref,
               s_out_ref, logits_ref,
               m_ref, d_ref, acc_ref):
    i = pl.program_id(0)

    @pl.when(i == 0)
    def _init():
        m_ref[0, 0] = _NEG
        d_ref[0, 0] = 0.0
        acc_ref[...] = jnp.zeros_like(acc_ref)

    hb = h_ref[...].astype(jnp.bfloat16)
    x = jnp.maximum(
        jnp.dot(hb, W1_ref[...], preferred_element_type=jnp.float32)
        + b1_ref[...], 0.0)
    xb = x.astype(jnp.bfloat16)
    a = jnp.tanh(
        jnp.dot(xb, Wa_ref[...], preferred_element_type=jnp.float32)
        + ba_ref[...])
    b = jax.nn.sigmoid(
        jnp.dot(xb, Wb_ref[...], preferred_element_type=jnp.float32)
        + bb_ref[...])
    g = a * b
    s = jnp.sum(g * wc_ref[...], axis=1, keepdims=True) + bc_ref[0, 0]
    s_out_ref[...] = s

    m_old = m_ref[0, 0]
    m_new = jnp.maximum(m_old, jnp.max(s))
    alpha = jnp.exp(m_old - m_new)
    p = jnp.exp(s - m_new)
    d_ref[0, 0] = d_ref[0, 0] * alpha + jnp.sum(p)
    px = jax.lax.dot_general(p, x, (((0,), (0,)), ((), ())),
                             preferred_element_type=jnp.float32)
    acc_ref[...] = acc_ref[...] * alpha + px
    m_ref[0, 0] = m_new

    @pl.when(i == pl.num_programs(0) - 1)
    def _fin():
        M = acc_ref[...] / d_ref[0, 0]
        logits_ref[...] = (
            jnp.dot(M, Wcls_ref[...], preferred_element_type=jnp.float32)
            + bcls_ref[...])


def _run_main(h, W1, b1, Wa, ba, Wb, bb, Wc, bc, Wcls, bcls, block_n):
    N, F = h.shape
    D1 = W1.shape[1]
    grid = N // block_n
    full = lambda shp: pl.BlockSpec(shp, lambda i: (0, 0))
    s, logits = pl.pallas_call(
        _main_body,
        grid=(grid,),
        in_specs=[
            pl.BlockSpec((block_n, F), lambda i: (i, 0)),
            full(W1.shape), full((1, D1)),
            full(Wa.shape), full((1, Wa.shape[1])),
            full(Wb.shape), full((1, Wb.shape[1])),
            full((1, Wc.shape[0])), full((1, 1)),
            full(Wcls.shape), full((1, 2)),
        ],
        out_specs=[
            pl.BlockSpec((block_n, 1), lambda i: (i, 0)),
            pl.BlockSpec((1, 2), lambda i: (0, 0)),
        ],
        out_shape=[
            jax.ShapeDtypeStruct((N, 1), jnp.float32),
            jax.ShapeDtypeStruct((1, 2), jnp.float32),
        ],
        scratch_shapes=[
            pltpu.SMEM((1, 1), jnp.float32),
            pltpu.SMEM((1, 1), jnp.float32),
            pltpu.VMEM((1, D1), jnp.float32),
        ],
        interpret=_INTERPRET,
    )(h, W1.astype(jnp.bfloat16), b1.reshape(1, -1),
      Wa.astype(jnp.bfloat16), ba.reshape(1, -1),
      Wb.astype(jnp.bfloat16), bb.reshape(1, -1),
      Wc.reshape(1, -1), bc.reshape(1, 1), Wcls, bcls.reshape(1, -1))
    return s, logits


def _topk_body(s_ref, ids_ref, t_ref, b_ref):
    R, C = s_ref.shape
    s = s_ref[...]
    gidx = (jax.lax.broadcasted_iota(jnp.int32, (R, C), 0) * C
            + jax.lax.broadcasted_iota(jnp.int32, (R, C), 1))
    big = jnp.int32(2147483647)
    t_ref[...] = s
    b_ref[...] = s
    for k in range(8):
        v = t_ref[...]
        mx = jnp.max(v)
        sel = jnp.min(jnp.where(v == mx, gidx, big))
        ids_ref[k] = sel
        t_ref[...] = jnp.where(gidx == sel, _NEG, v)
    for k in range(8):
        v = b_ref[...]
        mn = jnp.min(v)
        sel = jnp.min(jnp.where(v == mn, gidx, big))
        ids_ref[8 + k] = sel
        b_ref[...] = jnp.where(gidx == sel, jnp.inf, v)


def _run_topk(s2d):
    return pl.pallas_call(
        _topk_body,
        in_specs=[pl.BlockSpec(memory_space=pltpu.VMEM)],
        out_specs=pl.BlockSpec(memory_space=pltpu.SMEM),
        out_shape=jax.ShapeDtypeStruct((16,), jnp.int32),
        scratch_shapes=[
            pltpu.VMEM(s2d.shape, jnp.float32),
            pltpu.VMEM(s2d.shape, jnp.float32),
        ],
        interpret=_INTERPRET,
    )(s2d)


_SC_TILES = 16


def _merge16(Tv, Ti, v, ids):
    vs, is_ = plsc.sort_key_val(v, ids, descending=True)
    vr = lax.rev(vs, (0,))
    ir = lax.rev(is_, (0,))
    take = Tv >= vr
    hv = jnp.where(take, Tv, vr)
    hi = jnp.where(take, Ti, ir)
    return plsc.sort_key_val(hv, hi, descending=True)


def _merge16_sorted(Tv, Ti, v, ids):
    vr = lax.rev(v, (0,))
    ir = lax.rev(ids, (0,))
    take = Tv >= vr
    hv = jnp.where(take, Tv, vr)
    hi = jnp.where(take, Ti, ir)
    return plsc.sort_key_val(hv, hi, descending=True)


def _sc_body(n_total, chunk, sp_hbm, h_hbm, rows_out, ids_out,
             sh_tv, sh_ti, sh_bv, sh_bi,
             chunk_ref, st_tv, st_ti, st_bv, st_bi,
             all_tv, all_ti, all_bv, all_bi, idx_ref, rows_ref, sem):
    wid = lax.axis_index("s")
    base = wid * chunk
    pltpu.sync_copy(sp_hbm.at[pl.ds(base, chunk)], chunk_ref)

    lane = lax.broadcasted_iota(jnp.int32, (16,), 0)
    neg = jnp.full((16,), _NEG, jnp.float32)
    izero = jnp.zeros((16,), jnp.int32)

    def step(j, carry):
        tv, ti, bv, bi = carry
        v = chunk_ref[pl.ds(j * 16, 16)]
        gidx = base + j * 16 + lane
        valid = gidx < n_total
        vt = jnp.where(valid, v, _NEG)
        vn = jnp.where(valid, -v, _NEG)
        tv, ti = _merge16(tv, ti, vt, gidx)
        bv, bi = _merge16(bv, bi, vn, gidx)
        return tv, ti, bv, bi

    tv, ti, bv, bi = lax.fori_loop(0, chunk // 16, step,
                                   (neg, izero, neg, izero))

    st_tv[...] = tv
    st_ti[...] = ti
    st_bv[...] = bv
    st_bi[...] = bi
    pltpu.sync_copy(st_tv, sh_tv.at[wid])
    pltpu.sync_copy(st_ti, sh_ti.at[wid])
    pltpu.sync_copy(st_bv, sh_bv.at[wid])
    pltpu.sync_copy(st_bi, sh_bi.at[wid])
    plsc.subcore_barrier()

    @pl.when(wid == 0)
    def _final():
        pltpu.sync_copy(sh_tv, all_tv)
        pltpu.sync_copy(sh_ti, all_ti)
        pltpu.sync_copy(sh_bv, all_bv)
        pltpu.sync_copy(sh_bi, all_bi)
        gtv, gti, gbv, gbi = tv, ti, bv, bi
        for c in range(1, _SC_TILES):
            gtv, gti = _merge16_sorted(gtv, gti, all_tv[c], all_ti[c])
            gbv, gbi = _merge16_sorted(gbv, gbi, all_bv[c], all_bi[c])
        idx_ref[pl.ds(0, 16)] = gti
        idx_ref[pl.ds(16, 16)] = gbi
        pltpu.sync_copy(idx_ref, ids_out)
        pltpu.async_copy(h_hbm.at[idx_ref], rows_ref, sem).wait()
        pltpu.sync_copy(rows_ref, rows_out)


def _run_sc_topk_gather(sp, h):
    n_total, feat = h.shape
    npad = sp.shape[0]
    chunk = npad // _SC_TILES
    mesh = plsc.VectorSubcoreMesh(core_axis_name="c", subcore_axis_name="s",
                                  num_cores=1)
    f = pl.kernel(
        functools.partial(_sc_body, n_total, chunk),
        out_type=[
            jax.ShapeDtypeStruct((32, feat), jnp.float32),
            jax.ShapeDtypeStruct((32,), jnp.int32),
            jax.ShapeDtypeStruct((_SC_TILES, 16), jnp.float32),
            jax.ShapeDtypeStruct((_SC_TILES, 16), jnp.int32),
            jax.ShapeDtypeStruct((_SC_TILES, 16), jnp.float32),
            jax.ShapeDtypeStruct((_SC_TILES, 16), jnp.int32),
        ],
        compiler_params=pltpu.CompilerParams(needs_layout_passes=False),
        mesh=mesh,
        scratch_types=[
            pltpu.VMEM((chunk,), jnp.float32),
            pltpu.VMEM((16,), jnp.float32),
            pltpu.VMEM((16,), jnp.int32),
            pltpu.VMEM((16,), jnp.float32),
            pltpu.VMEM((16,), jnp.int32),
            pltpu.VMEM((_SC_TILES, 16), jnp.float32),
            pltpu.VMEM((_SC_TILES, 16), jnp.int32),
            pltpu.VMEM((_SC_TILES, 16), jnp.float32),
            pltpu.VMEM((_SC_TILES, 16), jnp.int32),
            pltpu.VMEM((32,), jnp.int32),
            pltpu.VMEM((32, feat), jnp.float32),
            pltpu.SemaphoreType.DMA,
        ],
    )
    rows, ids, _, _, _, _ = f(sp, h)
    return rows, ids


def _loss32_body(rows_ref, W1_ref, b1_ref, Wa_ref, ba_ref, Wb_ref, bb_ref,
                 wc_ref, bc_ref, Wi_ref, bi_ref, loss_ref):
    X = jnp.maximum(
        jnp.dot(rows_ref[...], W1_ref[...], preferred_element_type=jnp.float32)
        + b1_ref[...], 0.0)
    a = jnp.tanh(
        jnp.dot(X, Wa_ref[...], preferred_element_type=jnp.float32)
        + ba_ref[...])
    b = jax.nn.sigmoid(
        jnp.dot(X, Wb_ref[...], preferred_element_type=jnp.float32)
        + bb_ref[...])
    se = (jnp.sum(a * b * wc_ref[...], axis=1, keepdims=True)
          + bc_ref[0, 0])
    z = jnp.dot(X, Wi_ref[...], preferred_element_type=jnp.float32) + bi_ref[...]
    m2 = jnp.max(z, axis=1, keepdims=True)
    lse = m2 + jnp.log(jnp.sum(jnp.exp(z - m2), axis=1, keepdims=True))
    lane = lax.broadcasted_iota(jnp.int32, (32, 2), 1)
    lp1 = jnp.sum(jnp.where(lane == 1, z, 0.0), axis=1, keepdims=True) - lse
    lp0 = jnp.sum(jnp.where(lane == 0, z, 0.0), axis=1, keepdims=True) - lse
    row = lax.broadcasted_iota(jnp.int32, (32, 1), 0)
    big = jnp.int32(2147483647)

    st = jnp.where(row < 16, se, _NEG)
    msel = row < 0
    for _ in range(8):
        mx = jnp.max(st)
        sel = jnp.min(jnp.where(st == mx, row, big))
        msel = jnp.logical_or(msel, row == sel)
        st = jnp.where(row == sel, _NEG, st)
    sb = jnp.where(row >= 16, se, jnp.inf)
    nsel = row < 0
    for _ in range(8):
        mn = jnp.min(sb)
        sel = jnp.min(jnp.where(sb == mn, row, big))
        nsel = jnp.logical_or(nsel, row == sel)
        sb = jnp.where(row == sel, jnp.inf, sb)

    tot = (jnp.sum(jnp.where(msel, lp1, 0.0))
           + jnp.sum(jnp.where(nsel, lp0, 0.0)))
    loss_ref[0, 0] = -tot * (1.0 / 16.0)


def _run_loss32(rows, W1, b1, Wa, ba, Wb, bb, Wc, bc, Winst, binst):
    return pl.pallas_call(
        _loss32_body,
        out_specs=pl.BlockSpec(memory_space=pltpu.SMEM),
        out_shape=jax.ShapeDtypeStruct((1, 1), jnp.float32),
        interpret=_INTERPRET,
    )(rows, W1, b1.reshape(1, -1), Wa, ba.reshape(1, -1), Wb,
      bb.reshape(1, -1), Wc.reshape(1, -1), bc.reshape(1, 1), Winst,
      binst.reshape(1, -1))


def _loss_body(ids_ref, h_ref, W1_ref, b1_ref, Wi_ref, bi_ref,
               loss_ref, l_ref):
    i = pl.program_id(0)

    @pl.when(i == 0)
    def _init():
        l_ref[0, 0] = 0.0

    hrow = h_ref[...].reshape(1, h_ref.shape[2])
    x = jnp.maximum(
        jnp.dot(hrow, W1_ref[...], preferred_element_type=jnp.float32)
        + b1_ref[...], 0.0)
    z = jnp.dot(x, Wi_ref[...], preferred_element_type=jnp.float32) + bi_ref[...]
    m2 = jnp.max(z)
    lse = m2 + jnp.log(jnp.sum(jnp.exp(z - m2)))
    lane = jax.lax.broadcasted_iota(jnp.int32, (1, 2), 1)
    tgt = jnp.where(i < 8, 1, 0)
    zt = jnp.sum(jnp.where(lane == tgt, z, 0.0))
    l_ref[0, 0] = l_ref[0, 0] - (zt - lse) * (1.0 / 16.0)

    @pl.when(i == pl.num_programs(0) - 1)
    def _fin():
        loss_ref[0, 0] = l_ref[0, 0]


def _run_loss(ids, h, W1, b1, Winst, binst):
    F = h.shape[1]
    D1 = W1.shape[1]
    grid_spec = pltpu.PrefetchScalarGridSpec(
        num_scalar_prefetch=1,
        grid=(16,),
        in_specs=[
            pl.BlockSpec((1, 1, F), lambda i, ids: (ids[i], 0, 0)),
            pl.BlockSpec((F, D1), lambda i, ids: (0, 0)),
            pl.BlockSpec((1, D1), lambda i, ids: (0, 0)),
            pl.BlockSpec((D1, 2), lambda i, ids: (0, 0)),
            pl.BlockSpec((1, 2), lambda i, ids: (0, 0)),
        ],
        out_specs=pl.BlockSpec(memory_space=pltpu.SMEM),
        scratch_shapes=[pltpu.SMEM((1, 1), jnp.float32)],
    )
    return pl.pallas_call(
        _loss_body,
        grid_spec=grid_spec,
        out_shape=jax.ShapeDtypeStruct((1, 1), jnp.float32),
        interpret=_INTERPRET,
    )(ids, h.reshape(h.shape[0], 1, F), W1, b1.reshape(1, -1), Winst,
      binst.reshape(1, -1))


@jax.jit
def kernel(h, label, W1, b1, Wa, ba, Wb, bb, Wc, bc, Wcls, bcls, Winst, binst):
    N, F = h.shape
    block_n = 2000
    s, logits = _run_main(h, W1, b1, Wa, ba, Wb, bb, Wc, bc, Wcls, bcls,
                          block_n)
    npad = ((N + 16 * _SC_TILES - 1) // (16 * _SC_TILES)) * (16 * _SC_TILES)
    sp = jnp.pad(s.reshape(-1), (0, npad - N))
    rows, _ids = _run_sc_topk_gather(sp, h)
    loss = _run_loss32(rows, W1, b1, Wa, ba, Wb, bb, Wc, bc, Winst, binst)
    return logits, loss.reshape(())


# speedup vs baseline: 1.0343x; 1.0343x over previous
import functools

import jax
import jax.numpy as jnp
from jax import lax
from jax.experimental import pallas as pl
from jax.experimental.pallas import tpu as pltpu
from jax.experimental.pallas import tpu_sc as plsc

_INTERPRET = False

_NEG = float("-inf")


def _main_body(h_ref, W1_ref, b1_ref, Wa_ref, ba_ref, Wb_ref, bb_ref,
               wc_ref, bc_ref, Wcls_ref, bcls_ref,
               s_out_ref, logits_ref,
               m_ref, d_ref, acc_ref):
    i = pl.program_id(0)

    @pl.when(i == 0)
    def _init():
        m_ref[0, 0] = _NEG
        d_ref[0, 0] = 0.0
        acc_ref[...] = jnp.zeros_like(acc_ref)

    x = jnp.maximum(
        jnp.dot(h_ref[...], W1_ref[...], preferred_element_type=jnp.float32)
        + b1_ref[...], 0.0)
    a = jnp.tanh(
        jnp.dot(x, Wa_ref[...], preferred_element_type=jnp.float32)
        + ba_ref[...])
    b = jax.nn.sigmoid(
        jnp.dot(x, Wb_ref[...], preferred_element_type=jnp.float32)
        + bb_ref[...])
    g = a * b
    s = jnp.sum(g * wc_ref[...], axis=1, keepdims=True) + bc_ref[0, 0]
    s_out_ref[...] = s

    m_old = m_ref[0, 0]
    m_new = jnp.maximum(m_old, jnp.max(s))
    alpha = jnp.exp(m_old - m_new)
    p = jnp.exp(s - m_new)
    d_ref[0, 0] = d_ref[0, 0] * alpha + jnp.sum(p)
    px = jax.lax.dot_general(p, x, (((0,), (0,)), ((), ())),
                             preferred_element_type=jnp.float32)
    acc_ref[...] = acc_ref[...] * alpha + px
    m_ref[0, 0] = m_new

    @pl.when(i == pl.num_programs(0) - 1)
    def _fin():
        M = acc_ref[...] / d_ref[0, 0]
        logits_ref[...] = (
            jnp.dot(M, Wcls_ref[...], preferred_element_type=jnp.float32)
            + bcls_ref[...])


def _run_main(h, W1, b1, Wa, ba, Wb, bb, Wc, bc, Wcls, bcls, block_n):
    N, F = h.shape
    D1 = W1.shape[1]
    grid = N // block_n
    full = lambda shp: pl.BlockSpec(shp, lambda i: (0, 0))
    s, logits = pl.pallas_call(
        _main_body,
        grid=(grid,),
        in_specs=[
            pl.BlockSpec((block_n, F), lambda i: (i, 0)),
            full(W1.shape), full((1, D1)),
            full(Wa.shape), full((1, Wa.shape[1])),
            full(Wb.shape), full((1, Wb.shape[1])),
            full((1, Wc.shape[0])), full((1, 1)),
            full(Wcls.shape), full((1, 2)),
        ],
        out_specs=[
            pl.BlockSpec((block_n, 1), lambda i: (i, 0)),
            pl.BlockSpec((1, 2), lambda i: (0, 0)),
        ],
        out_shape=[
            jax.ShapeDtypeStruct((N, 1), jnp.float32),
            jax.ShapeDtypeStruct((1, 2), jnp.float32),
        ],
        scratch_shapes=[
            pltpu.SMEM((1, 1), jnp.float32),
            pltpu.SMEM((1, 1), jnp.float32),
            pltpu.VMEM((1, D1), jnp.float32),
        ],
        interpret=_INTERPRET,
    )(h, W1, b1.reshape(1, -1), Wa, ba.reshape(1, -1), Wb, bb.reshape(1, -1),
      Wc.reshape(1, -1), bc.reshape(1, 1), Wcls, bcls.reshape(1, -1))
    return s, logits


def _topk_body(s_ref, ids_ref, t_ref, b_ref):
    R, C = s_ref.shape
    s = s_ref[...]
    gidx = (jax.lax.broadcasted_iota(jnp.int32, (R, C), 0) * C
            + jax.lax.broadcasted_iota(jnp.int32, (R, C), 1))
    big = jnp.int32(2147483647)
    t_ref[...] = s
    b_ref[...] = s
    for k in range(8):
        v = t_ref[...]
        mx = jnp.max(v)
        sel = jnp.min(jnp.where(v == mx, gidx, big))
        ids_ref[k] = sel
        t_ref[...] = jnp.where(gidx == sel, _NEG, v)
    for k in range(8):
        v = b_ref[...]
        mn = jnp.min(v)
        sel = jnp.min(jnp.where(v == mn, gidx, big))
        ids_ref[8 + k] = sel
        b_ref[...] = jnp.where(gidx == sel, jnp.inf, v)


def _run_topk(s2d):
    return pl.pallas_call(
        _topk_body,
        in_specs=[pl.BlockSpec(memory_space=pltpu.VMEM)],
        out_specs=pl.BlockSpec(memory_space=pltpu.SMEM),
        out_shape=jax.ShapeDtypeStruct((16,), jnp.int32),
        scratch_shapes=[
            pltpu.VMEM(s2d.shape, jnp.float32),
            pltpu.VMEM(s2d.shape, jnp.float32),
        ],
        interpret=_INTERPRET,
    )(s2d)


_SC_TILES = 16


def _merge16(Tv, Ti, v, ids):
    vs, is_ = plsc.sort_key_val(v, ids, descending=True)
    vr = lax.rev(vs, (0,))
    ir = lax.rev(is_, (0,))
    take = Tv >= vr
    hv = jnp.where(take, Tv, vr)
    hi = jnp.where(take, Ti, ir)
    return plsc.sort_key_val(hv, hi, descending=True)


def _merge16_sorted(Tv, Ti, v, ids):
    vr = lax.rev(v, (0,))
    ir = lax.rev(ids, (0,))
    take = Tv >= vr
    hv = jnp.where(take, Tv, vr)
    hi = jnp.where(take, Ti, ir)
    return plsc.sort_key_val(hv, hi, descending=True)


def _sc_body(n_total, chunk, sp_hbm, h_hbm, rows_out, ids_out,
             sh_tv, sh_ti, sh_bv, sh_bi,
             chunk_ref, st_tv, st_ti, st_bv, st_bi,
             all_tv, all_ti, all_bv, all_bi, idx_ref, rows_ref, sem):
    wid = lax.axis_index("s")
    base = wid * chunk
    pltpu.sync_copy(sp_hbm.at[pl.ds(base, chunk)], chunk_ref)

    lane = lax.broadcasted_iota(jnp.int32, (16,), 0)
    neg = jnp.full((16,), _NEG, jnp.float32)
    izero = jnp.zeros((16,), jnp.int32)

    def step(j, carry):
        tv, ti, bv, bi = carry
        v = chunk_ref[pl.ds(j * 16, 16)]
        gidx = base + j * 16 + lane
        valid = gidx < n_total
        vt = jnp.where(valid, v, _NEG)
        vn = jnp.where(valid, -v, _NEG)
        tv, ti = _merge16(tv, ti, vt, gidx)
        bv, bi = _merge16(bv, bi, vn, gidx)
        return tv, ti, bv, bi

    tv, ti, bv, bi = lax.fori_loop(0, chunk // 16, step,
                                   (neg, izero, neg, izero))

    st_tv[...] = tv
    st_ti[...] = ti
    st_bv[...] = bv
    st_bi[...] = bi
    pltpu.sync_copy(st_tv, sh_tv.at[wid])
    pltpu.sync_copy(st_ti, sh_ti.at[wid])
    pltpu.sync_copy(st_bv, sh_bv.at[wid])
    pltpu.sync_copy(st_bi, sh_bi.at[wid])
    plsc.subcore_barrier()

    @pl.when(wid == 0)
    def _final():
        pltpu.sync_copy(sh_tv, all_tv)
        pltpu.sync_copy(sh_ti, all_ti)
        pltpu.sync_copy(sh_bv, all_bv)
        pltpu.sync_copy(sh_bi, all_bi)
        gtv, gti, gbv, gbi = tv, ti, bv, bi
        for c in range(1, _SC_TILES):
            gtv, gti = _merge16_sorted(gtv, gti, all_tv[c], all_ti[c])
            gbv, gbi = _merge16_sorted(gbv, gbi, all_bv[c], all_bi[c])
        idx_ref[pl.ds(0, 16)] = gti
        idx_ref[pl.ds(16, 16)] = gbi
        pltpu.sync_copy(idx_ref, ids_out)
        pltpu.async_copy(h_hbm.at[idx_ref], rows_ref, sem).wait()
        pltpu.sync_copy(rows_ref, rows_out)


def _run_sc_topk_gather(sp, h):
    n_total, feat = h.shape
    npad = sp.shape[0]
    chunk = npad // _SC_TILES
    mesh = plsc.VectorSubcoreMesh(core_axis_name="c", subcore_axis_name="s",
                                  num_cores=1)
    f = pl.kernel(
        functools.partial(_sc_body, n_total, chunk),
        out_type=[
            jax.ShapeDtypeStruct((32, feat), jnp.float32),
            jax.ShapeDtypeStruct((32,), jnp.int32),
            jax.ShapeDtypeStruct((_SC_TILES, 16), jnp.float32),
            jax.ShapeDtypeStruct((_SC_TILES, 16), jnp.int32),
            jax.ShapeDtypeStruct((_SC_TILES, 16), jnp.float32),
            jax.ShapeDtypeStruct((_SC_TILES, 16), jnp.int32),
        ],
        compiler_params=pltpu.CompilerParams(needs_layout_passes=False),
        mesh=mesh,
        scratch_types=[
            pltpu.VMEM((chunk,), jnp.float32),
            pltpu.VMEM((16,), jnp.float32),
            pltpu.VMEM((16,), jnp.int32),
            pltpu.VMEM((16,), jnp.float32),
            pltpu.VMEM((16,), jnp.int32),
            pltpu.VMEM((_SC_TILES, 16), jnp.float32),
            pltpu.VMEM((_SC_TILES, 16), jnp.int32),
            pltpu.VMEM((_SC_TILES, 16), jnp.float32),
            pltpu.VMEM((_SC_TILES, 16), jnp.int32),
            pltpu.VMEM((32,), jnp.int32),
            pltpu.VMEM((32, feat), jnp.float32),
            pltpu.SemaphoreType.DMA,
        ],
    )
    rows, ids, _, _, _, _ = f(sp, h)
    return rows, ids


def _loss32_body(rows_ref, W1_ref, b1_ref, Wa_ref, ba_ref, Wb_ref, bb_ref,
                 wc_ref, bc_ref, Wi_ref, bi_ref, loss_ref):
    X = jnp.maximum(
        jnp.dot(rows_ref[...], W1_ref[...], preferred_element_type=jnp.float32)
        + b1_ref[...], 0.0)
    a = jnp.tanh(
        jnp.dot(X, Wa_ref[...], preferred_element_type=jnp.float32)
        + ba_ref[...])
    b = jax.nn.sigmoid(
        jnp.dot(X, Wb_ref[...], preferred_element_type=jnp.float32)
        + bb_ref[...])
    se = (jnp.sum(a * b * wc_ref[...], axis=1, keepdims=True)
          + bc_ref[0, 0])
    z = jnp.dot(X, Wi_ref[...], preferred_element_type=jnp.float32) + bi_ref[...]
    m2 = jnp.max(z, axis=1, keepdims=True)
    lse = m2 + jnp.log(jnp.sum(jnp.exp(z - m2), axis=1, keepdims=True))
    lane = lax.broadcasted_iota(jnp.int32, (32, 2), 1)
    lp1 = jnp.sum(jnp.where(lane == 1, z, 0.0), axis=1, keepdims=True) - lse
    lp0 = jnp.sum(jnp.where(lane == 0, z, 0.0), axis=1, keepdims=True) - lse
    row = lax.broadcasted_iota(jnp.int32, (32, 1), 0)
    big = jnp.int32(2147483647)

    st = jnp.where(row < 16, se, _NEG)
    msel = row < 0
    for _ in range(8):
        mx = jnp.max(st)
        sel = jnp.min(jnp.where(st == mx, row, big))
        msel = jnp.logical_or(msel, row == sel)
        st = jnp.where(row == sel, _NEG, st)
    sb = jnp.where(row >= 16, se, jnp.inf)
    nsel = row < 0
    for _ in range(8):
        mn = jnp.min(sb)
        sel = jnp.min(jnp.where(sb == mn, row, big))
        nsel = jnp.logical_or(nsel, row == sel)
        sb = jnp.where(row == sel, jnp.inf, sb)

    tot = (jnp.sum(jnp.where(msel, lp1, 0.0))
           + jnp.sum(jnp.where(nsel, lp0, 0.0)))
    loss_ref[0, 0] = -tot * (1.0 / 16.0)


def _run_loss32(rows, W1, b1, Wa, ba, Wb, bb, Wc, bc, Winst, binst):
    return pl.pallas_call(
        _loss32_body,
        out_specs=pl.BlockSpec(memory_space=pltpu.SMEM),
        out_shape=jax.ShapeDtypeStruct((1, 1), jnp.float32),
        interpret=_INTERPRET,
    )(rows, W1, b1.reshape(1, -1), Wa, ba.reshape(1, -1), Wb,
      bb.reshape(1, -1), Wc.reshape(1, -1), bc.reshape(1, 1), Winst,
      binst.reshape(1, -1))


def _loss_body(ids_ref, h_ref, W1_ref, b1_ref, Wi_ref, bi_ref,
               loss_ref, l_ref):
    i = pl.program_id(0)

    @pl.when(i == 0)
    def _init():
        l_ref[0, 0] = 0.0

    hrow = h_ref[...].reshape(1, h_ref.shape[2])
    x = jnp.maximum(
        jnp.dot(hrow, W1_ref[...], preferred_element_type=jnp.float32)
        + b1_ref[...], 0.0)
    z = jnp.dot(x, Wi_ref[...], preferred_element_type=jnp.float32) + bi_ref[...]
    m2 = jnp.max(z)
    lse = m2 + jnp.log(jnp.sum(jnp.exp(z - m2)))
    lane = jax.lax.broadcasted_iota(jnp.int32, (1, 2), 1)
    tgt = jnp.where(i < 8, 1, 0)
    zt = jnp.sum(jnp.where(lane == tgt, z, 0.0))
    l_ref[0, 0] = l_ref[0, 0] - (zt - lse) * (1.0 / 16.0)

    @pl.when(i == pl.num_programs(0) - 1)
    def _fin():
        loss_ref[0, 0] = l_ref[0, 0]


def _run_loss(ids, h, W1, b1, Winst, binst):
    F = h.shape[1]
    D1 = W1.shape[1]
    grid_spec = pltpu.PrefetchScalarGridSpec(
        num_scalar_prefetch=1,
        grid=(16,),
        in_specs=[
            pl.BlockSpec((1, 1, F), lambda i, ids: (ids[i], 0, 0)),
            pl.BlockSpec((F, D1), lambda i, ids: (0, 0)),
            pl.BlockSpec((1, D1), lambda i, ids: (0, 0)),
            pl.BlockSpec((D1, 2), lambda i, ids: (0, 0)),
            pl.BlockSpec((1, 2), lambda i, ids: (0, 0)),
        ],
        out_specs=pl.BlockSpec(memory_space=pltpu.SMEM),
        scratch_shapes=[pltpu.SMEM((1, 1), jnp.float32)],
    )
    return pl.pallas_call(
        _loss_body,
        grid_spec=grid_spec,
        out_shape=jax.ShapeDtypeStruct((1, 1), jnp.float32),
        interpret=_INTERPRET,
    )(ids, h.reshape(h.shape[0], 1, F), W1, b1.reshape(1, -1), Winst,
      binst.reshape(1, -1))


@jax.jit
def kernel(h, label, W1, b1, Wa, ba, Wb, bb, Wc, bc, Wcls, bcls, Winst, binst):
    N, F = h.shape
    block_n = 4000
    s, logits = _run_main(h, W1, b1, Wa, ba, Wb, bb, Wc, bc, Wcls, bcls,
                          block_n)
    npad = ((N + 16 * _SC_TILES - 1) // (16 * _SC_TILES)) * (16 * _SC_TILES)
    sp = jnp.pad(s.reshape(-1), (0, npad - N))
    rows, _ids = _run_sc_topk_gather(sp, h)
    loss = _run_loss32(rows, W1, b1, Wa, ba, Wb, bb, Wc, bc, Winst, binst)
    return logits, loss.reshape(())
